# Initial kernel scaffold; baseline (speedup 1.0000x reference)
import jax
import jax.numpy as jnp
from jax.experimental import pallas as pl


def kernel(z, edge_index):
    raise NotImplementedError("write your pallas kernel here")


# speedup vs baseline: 2.6054x; 2.6054x over previous
import jax
import jax.numpy as jnp
from jax import lax
from jax.experimental import pallas as pl
from jax.experimental.pallas import tpu as pltpu
from jax.experimental.pallas import tpu_sc as plsc

_NC = 2
_NS = 16
_NW = _NC * _NS
_LANES = 16

_E = 320000
_D = 128
_VPR = _D // _LANES
_CHUNK = 128
_NGLOBAL = _E // _CHUNK
_STEPS = -(-_NGLOBAL // _NW)

_SHUF_DNUMS = lax.GatherDimensionNumbers(
    offset_dims=(), collapsed_slice_dims=(0,), start_index_map=(0,))


def _shuffle(v, perm):
    return lax.gather(v, perm[:, None], _SHUF_DNUMS, slice_sizes=(1,),
                      mode=lax.GatherScatterMode.PROMISE_IN_BOUNDS)


def _sc_body(z_hbm, src_hbm, dst_hbm, out_hbm,
             src_idx_v, dst_idx_v, src_rows_v, dst_rows_v, sem_s, sem_d):
    wid = lax.axis_index("s") * _NC + lax.axis_index("c")

    def chunk_body(i, carry):
        g = wid + i * _NW

        @pl.when(g < _NGLOBAL)
        def _():
            base = g * _CHUNK
            pltpu.sync_copy(src_hbm.at[pl.ds(base, _CHUNK)], src_idx_v)
            pltpu.sync_copy(dst_hbm.at[pl.ds(base, _CHUNK)], dst_idx_v)
            cp_s = pltpu.async_copy(z_hbm.at[src_idx_v], src_rows_v, sem_s)
            cp_d = pltpu.async_copy(z_hbm.at[dst_idx_v], dst_rows_v, sem_d)
            cp_s.wait()
            cp_d.wait()

            def edge_body(e, carry2):
                v = [src_rows_v[e, pl.ds(j * _LANES, _LANES)]
                     * dst_rows_v[e, pl.ds(j * _LANES, _LANES)]
                     for j in range(_VPR)]
                m = v[0]
                for j in range(1, _VPR):
                    m = jnp.maximum(m, v[j])
                for sh in (8, 4, 2, 1):
                    perm = lax.iota(jnp.int32, 16) ^ sh
                    m = jnp.maximum(m, _shuffle(m, perm))
                ex = [jnp.exp(v[j] - m) for j in range(_VPR)]
                s = ex[0]
                for j in range(1, _VPR):
                    s = s + ex[j]
                for sh in (8, 4, 2, 1):
                    perm = lax.iota(jnp.int32, 16) ^ sh
                    s = s + _shuffle(s, perm)
                r = 1.0 / s
                for j in range(_VPR):
                    src_rows_v[e, pl.ds(j * _LANES, _LANES)] = ex[j] * r
                return carry2

            lax.fori_loop(0, _CHUNK, edge_body, 0)
            pltpu.sync_copy(src_rows_v, out_hbm.at[pl.ds(base, _CHUNK)])

        return carry

    lax.fori_loop(0, _STEPS, chunk_body, 0)


def _decode(z, src_idx, dst_idx):
    mesh = plsc.VectorSubcoreMesh(core_axis_name="c", subcore_axis_name="s",
                                  num_cores=_NC, num_subcores=_NS)
    return pl.kernel(
        _sc_body,
        out_type=jax.ShapeDtypeStruct((_E, _D), jnp.float32),
        mesh=mesh,
        scratch_types=[
            pltpu.VMEM((_CHUNK,), jnp.int32),
            pltpu.VMEM((_CHUNK,), jnp.int32),
            pltpu.VMEM((_CHUNK, _D), jnp.float32),
            pltpu.VMEM((_CHUNK, _D), jnp.float32),
            pltpu.SemaphoreType.DMA,
            pltpu.SemaphoreType.DMA,
        ],
    )(z, src_idx, dst_idx)


def kernel(z, edge_index):
    ei = edge_index.astype(jnp.int32)
    return _decode(z, ei[0], ei[1])


# speedup vs baseline: 4.4570x; 1.7107x over previous
import jax
import jax.numpy as jnp
from jax import lax
from jax.experimental import pallas as pl
from jax.experimental.pallas import tpu as pltpu
from jax.experimental.pallas import tpu_sc as plsc

_NC = 2
_NS = 16
_NW = _NC * _NS
_LANES = 16

_E = 320000
_D = 128
_VPR = _D // _LANES
_CHUNK = 128
_NGLOBAL = _E // _CHUNK
_STEPS = -(-_NGLOBAL // _NW)

_SHUF_DNUMS = lax.GatherDimensionNumbers(
    offset_dims=(), collapsed_slice_dims=(0,), start_index_map=(0,))


def _shuffle(v, perm):
    return lax.gather(v, perm[:, None], _SHUF_DNUMS, slice_sizes=(1,),
                      mode=lax.GatherScatterMode.PROMISE_IN_BOUNDS)


def _sc_body(z_hbm, src_hbm, dst_hbm, out_hbm,
             src_idx_v, dst_idx_v, src_rows_v, dst_rows_v, sem_s, sem_d):
    wid = lax.axis_index("s") * _NC + lax.axis_index("c")

    def chunk_body(i, carry):
        g = wid + i * _NW

        @pl.when(g < _NGLOBAL)
        def _():
            base = g * _CHUNK
            pltpu.sync_copy(src_hbm.at[pl.ds(base, _CHUNK)], src_idx_v)
            pltpu.sync_copy(dst_hbm.at[pl.ds(base, _CHUNK)], dst_idx_v)
            cp_s = pltpu.async_copy(z_hbm.at[src_idx_v], src_rows_v, sem_s)
            cp_d = pltpu.async_copy(z_hbm.at[dst_idx_v], dst_rows_v, sem_d)
            cp_s.wait()
            cp_d.wait()

            @plsc.parallel_loop(0, _CHUNK, unroll=4)
            def edge_body(e):
                v = [src_rows_v[e, pl.ds(j * _LANES, _LANES)]
                     * dst_rows_v[e, pl.ds(j * _LANES, _LANES)]
                     for j in range(_VPR)]
                m = v[0]
                for j in range(1, _VPR):
                    m = jnp.maximum(m, v[j])
                for sh in (8, 4, 2, 1):
                    perm = lax.iota(jnp.int32, 16) ^ sh
                    m = jnp.maximum(m, _shuffle(m, perm))
                ex = [jnp.exp(v[j] - m) for j in range(_VPR)]
                s = ex[0]
                for j in range(1, _VPR):
                    s = s + ex[j]
                for sh in (8, 4, 2, 1):
                    perm = lax.iota(jnp.int32, 16) ^ sh
                    s = s + _shuffle(s, perm)
                r = 1.0 / s
                for j in range(_VPR):
                    src_rows_v[e, pl.ds(j * _LANES, _LANES)] = ex[j] * r

            pltpu.sync_copy(src_rows_v, out_hbm.at[pl.ds(base, _CHUNK)])

        return carry

    lax.fori_loop(0, _STEPS, chunk_body, 0)


def _decode(z, src_idx, dst_idx):
    mesh = plsc.VectorSubcoreMesh(core_axis_name="c", subcore_axis_name="s",
                                  num_cores=_NC, num_subcores=_NS)
    return pl.kernel(
        _sc_body,
        out_type=jax.ShapeDtypeStruct((_E, _D), jnp.float32),
        mesh=mesh,
        scratch_types=[
            pltpu.VMEM((_CHUNK,), jnp.int32),
            pltpu.VMEM((_CHUNK,), jnp.int32),
            pltpu.VMEM((_CHUNK, _D), jnp.float32),
            pltpu.VMEM((_CHUNK, _D), jnp.float32),
            pltpu.SemaphoreType.DMA,
            pltpu.SemaphoreType.DMA,
        ],
    )(z, src_idx, dst_idx)


def kernel(z, edge_index):
    ei = edge_index.astype(jnp.int32)
    return _decode(z, ei[0], ei[1])
